# baseline (speedup 1.0000x reference)
import jax
import jax.numpy as jnp
from jax import lax
from jax.experimental import pallas as pl
from jax.experimental.pallas import tpu as pltpu
from jax.experimental.pallas import tpu_sc as plsc

_OUT = 200
_N = 10000
_E = 320000
_NG = 64
_DIN = 128
_LAYERS = 6
_H = 128
_AGGW = 2 * _H

_NC = 2
_NS = 16
_K = 64
_CS = 32
_NST = 10
_EPT = _K * _CS * _NST
_ACC = 10016
_TRASH = 10008
_STR = 624

_BR = 1000
_GB = _N // _BR


def _sc_body(m_hbm, src_hbm, dst_hbm, z_hbm, out_hbm,
             src_a, dst_a, src_b, dst_b, r0, r1, r2, r3, acc,
             si, s0, s1, s2, s3, t0, t1, t2, t3):
    cid = lax.axis_index("c")
    sid = lax.axis_index("s")

    @pl.when(sid < _NS - 1)
    def _():
        pltpu.sync_copy(z_hbm.at[pl.ds(sid * _STR, _STR)],
                        acc.at[pl.ds(sid * _STR, _STR)])

    @pl.when(sid == _NS - 1)
    def _():
        base = (_NS - 1) * _STR
        pltpu.sync_copy(z_hbm.at[pl.ds(base, _ACC - base)],
                        acc.at[pl.ds(base, _ACC - base)])

    pltpu.sync_copy(src_hbm.at[sid, 0], src_a)
    pltpu.sync_copy(dst_hbm.at[sid, 0], dst_a)
    plsc.subcore_barrier()

    mh = m_hbm.at[cid]
    idx_bufs = ((src_a, dst_a), (src_b, dst_b))
    rows = (r0, r1, r2, r3)
    sems = (s0, s1, s2, s3)
    ssems = (t0, t1, t2, t3)

    for st in range(_NST):
        cur_s, cur_d = idx_bufs[st % 2]
        if st + 1 < _NST:
            nxt_s, nxt_d = idx_bufs[(st + 1) % 2]
            pltpu.async_copy(src_hbm.at[sid, st + 1], nxt_s, si)
            pltpu.async_copy(dst_hbm.at[sid, st + 1], nxt_d, si)

        for b in range(4):
            pltpu.async_copy(mh.at[cur_s.at[b]], rows[b], sems[b])

        @pl.loop(0, _CS - 4, step=4)
        def _(j, cur_s=cur_s, cur_d=cur_d):
            for b in range(4):
                pltpu.make_async_copy(mh.at[cur_s.at[j + b]],
                                      rows[b], sems[b]).wait()
                pltpu.async_copy(rows[b], acc.at[cur_d.at[j + b]], ssems[b],
                                 add=True)
            for b in range(4):
                pltpu.make_async_copy(rows[b], acc.at[cur_d.at[j + b]],
                                      ssems[b]).wait()
                pltpu.async_copy(mh.at[cur_s.at[j + b + 4]], rows[b], sems[b])

        for b in range(4):
            jj = _CS - 4 + b
            pltpu.make_async_copy(mh.at[cur_s.at[jj]],
                                  rows[b], sems[b]).wait()
            pltpu.sync_copy(rows[b], acc.at[cur_d.at[jj]], add=True)

        if st + 1 < _NST:
            pltpu.make_async_copy(src_hbm.at[sid, st + 1], nxt_s, si).wait()
            pltpu.make_async_copy(dst_hbm.at[sid, st + 1], nxt_d, si).wait()

    plsc.subcore_barrier()

    @pl.when(sid < _NS - 1)
    def _():
        pltpu.sync_copy(acc.at[pl.ds(sid * _STR, _STR)],
                        out_hbm.at[pl.ds(sid * _STR, _STR),
                                   pl.ds(cid * _H, _H)])

    @pl.when(sid == _NS - 1)
    def _():
        base = (_NS - 1) * _STR
        pltpu.sync_copy(acc.at[pl.ds(base, _N - base)],
                        out_hbm.at[pl.ds(base, _N - base),
                                   pl.ds(cid * _H, _H)])


def _sc_segment_sum(m3, src4, dst4, zeros):
    mesh = plsc.VectorSubcoreMesh(core_axis_name="c", subcore_axis_name="s")
    kern = pl.kernel(
        _sc_body,
        out_type=jax.ShapeDtypeStruct((_N, _AGGW), jnp.float32),
        mesh=mesh,
        scratch_types=[
            pltpu.VMEM((_CS, _K), jnp.int32),
            pltpu.VMEM((_CS, _K), jnp.int32),
            pltpu.VMEM((_CS, _K), jnp.int32),
            pltpu.VMEM((_CS, _K), jnp.int32),
            pltpu.VMEM((_K, _H), jnp.float32),
            pltpu.VMEM((_K, _H), jnp.float32),
            pltpu.VMEM((_K, _H), jnp.float32),
            pltpu.VMEM((_K, _H), jnp.float32),
            pltpu.VMEM_SHARED((_ACC, _H), jnp.float32),
            pltpu.SemaphoreType.DMA,
            pltpu.SemaphoreType.DMA,
            pltpu.SemaphoreType.DMA,
            pltpu.SemaphoreType.DMA,
            pltpu.SemaphoreType.DMA,
            pltpu.SemaphoreType.DMA,
            pltpu.SemaphoreType.DMA,
            pltpu.SemaphoreType.DMA,
            pltpu.SemaphoreType.DMA,
        ],
    )
    return kern(m3, src4, dst4, zeros)


def _dot(a, b):
    return jax.lax.dot_general(a, b, (((1,), (0,)), ((), ())),
                               preferred_element_type=jnp.float32)


def _full(shape):
    zeros = (0,) * len(shape)
    return pl.BlockSpec(shape, lambda i: zeros)


def _project0_body(x_ref, wa_ref, wb_ref, m_ref):
    x = x_ref[...]
    m_ref[0] = _dot(x, wa_ref[...])
    m_ref[1] = _dot(x, wb_ref[...])


def _project0(x, w0a, w0b):
    return pl.pallas_call(
        _project0_body,
        grid=(_GB,),
        in_specs=[pl.BlockSpec((_BR, _DIN), lambda i: (i, 0)),
                  _full((_DIN, _H)), _full((_DIN, _H))],
        out_specs=pl.BlockSpec((_NC, _BR, _H), lambda i: (0, i, 0)),
        out_shape=jax.ShapeDtypeStruct((_NC, _N, _H), jnp.float32),
        compiler_params=pltpu.CompilerParams(
            dimension_semantics=("arbitrary",)),
    )(x, w0a, w0b)


def _gru_math(p_ref, h_ref, wir, wiz, win, whr, whz, whn,
              bir, biz, bin_, bhr, bhz, bhn):
    agg = p_ref[...]
    h = h_ref[...]
    gir = _dot(agg, wir[...]) + bir[...]
    giz = _dot(agg, wiz[...]) + biz[...]
    gin = _dot(agg, win[...]) + bin_[...]
    ghr = _dot(h, whr[...]) + bhr[...]
    ghz = _dot(h, whz[...]) + bhz[...]
    ghn = _dot(h, whn[...]) + bhn[...]
    r = jax.nn.sigmoid(gir + ghr)
    z = jax.nn.sigmoid(giz + ghz)
    n = jnp.tanh(gin + r * ghn)
    return (1.0 - z) * n + z * h


def _gru_layer_body(p_ref, h_ref, wir, wiz, win, whr, whz, whn,
                    bir, biz, bin_, bhr, bhz, bhn, wna_ref, wnb_ref,
                    hout_ref, mout_ref):
    h_new = _gru_math(p_ref, h_ref, wir, wiz, win, whr, whz, whn,
                      bir, biz, bin_, bhr, bhz, bhn)
    hout_ref[...] = h_new
    mout_ref[0] = _dot(h_new, wna_ref[...])
    mout_ref[1] = _dot(h_new, wnb_ref[...])


def _gru_layer(p, h, mats, biases, w_next_a, w_next_b):
    in_specs = ([pl.BlockSpec((_BR, _AGGW), lambda i: (i, 0)),
                 pl.BlockSpec((_BR, _OUT), lambda i: (i, 0))]
                + [_full((_AGGW, _OUT))] * 3
                + [_full((_OUT, _OUT))] * 3
                + [_full((1, _OUT))] * 6
                + [_full((_OUT, _H))] * 2)
    return pl.pallas_call(
        _gru_layer_body,
        grid=(_GB,),
        in_specs=in_specs,
        out_specs=[pl.BlockSpec((_BR, _OUT), lambda i: (i, 0)),
                   pl.BlockSpec((_NC, _BR, _H), lambda i: (0, i, 0))],
        out_shape=[jax.ShapeDtypeStruct((_N, _OUT), jnp.float32),
                   jax.ShapeDtypeStruct((_NC, _N, _H), jnp.float32)],
        compiler_params=pltpu.CompilerParams(
            dimension_semantics=("arbitrary",)),
    )(p, h, *mats, *biases, w_next_a, w_next_b)


def _pool_body(h_ref, b_ref, cw_ref, cb_ref, o_ref, pooled_ref):
    i = pl.program_id(0)

    @pl.when(i == 0)
    def _():
        pooled_ref[...] = jnp.full((_NG, _OUT), -jnp.inf, jnp.float32)

    act = jnp.maximum(h_ref[...], 0.0)
    b = b_ref[...]
    g_lo = jnp.min(b)
    g_hi = jnp.max(b)

    def upd(g, _):
        mask = b == g
        mx = jnp.max(jnp.where(mask, act, -jnp.inf), axis=0, keepdims=True)
        cur = pooled_ref[pl.ds(g, 1), :]
        pooled_ref[pl.ds(g, 1), :] = jnp.maximum(cur, mx)
        return 0

    lax.fori_loop(g_lo, g_hi + 1, upd, 0)

    @pl.when(i == pl.num_programs(0) - 1)
    def _():
        logits = _dot(pooled_ref[...], cw_ref[...]) + cb_ref[...]
        o_ref[...] = jax.nn.sigmoid(logits)


def _pool(h, batch2, cls_wt, cls_b):
    in_specs = [pl.BlockSpec((_BR, _OUT), lambda i: (i, 0)),
                pl.BlockSpec((_BR, 1), lambda i: (i, 0)),
                _full((_OUT, 2)),
                _full((1, 2))]
    return pl.pallas_call(
        _pool_body,
        grid=(_GB,),
        in_specs=in_specs,
        out_specs=pl.BlockSpec((_NG, 2), lambda i: (0, 0)),
        out_shape=jax.ShapeDtypeStruct((_NG, 2), jnp.float32),
        scratch_shapes=[pltpu.VMEM((_NG, _OUT), jnp.float32)],
        compiler_params=pltpu.CompilerParams(
            dimension_semantics=("arbitrary",)),
    )(h, batch2, cls_wt, cls_b)


def _pad_cols(w, width):
    return jnp.pad(w, ((0, 0), (0, width - w.shape[1])))


def kernel(x, edge_index, batch, weight, w_ih, w_hh, b_ih, b_hh, cls_W, cls_b):
    pad = _EPT - _E // _NS
    src4 = jnp.pad(edge_index[0].reshape(_NS, _E // _NS),
                   ((0, 0), (0, pad))).reshape(_NS, _NST, _CS, _K)
    dst4 = jnp.pad(edge_index[1].reshape(_NS, _E // _NS),
                   ((0, 0), (0, pad)),
                   constant_values=_TRASH).reshape(_NS, _NST, _CS, _K)
    zeros = jnp.zeros((_ACC, _H), jnp.float32)
    h = jnp.pad(x, ((0, 0), (0, _OUT - _DIN)))

    wi = tuple(jnp.pad(w_ih[s * _OUT:(s + 1) * _OUT, :].T,
                       ((0, _AGGW - _OUT), (0, 0))) for s in range(3))
    wh = tuple(w_hh[s * _OUT:(s + 1) * _OUT, :].T for s in range(3))
    mats = wi + wh
    biases = tuple(bv[s * _OUT:(s + 1) * _OUT].reshape(1, _OUT)
                   for bv in (b_ih, b_hh) for s in range(3))
    batch2 = batch.reshape(_N, 1)
    cls_wt = cls_W.T
    cls_b2 = cls_b.reshape(1, 2)

    wa = [weight[i][:, 0:_H] for i in range(_LAYERS)]
    wb = [_pad_cols(weight[i][:, _H:_OUT], _H) for i in range(_LAYERS)]

    m = _project0(x, wa[0][0:_DIN], wb[0][0:_DIN])

    zw = jnp.zeros((_OUT, _H), jnp.float32)
    wa_next = jnp.stack(wa[1:] + [zw])
    wb_next = jnp.stack(wb[1:] + [zw])

    def step(carry, ws):
        h, m = carry
        w_next_a, w_next_b = ws
        p = _sc_segment_sum(m, src4, dst4, zeros)
        h, m = _gru_layer(p, h, mats, biases, w_next_a, w_next_b)
        return (h, m), None

    (h, m), _ = lax.scan(step, (h, m), (wa_next, wb_next))
    return _pool(h, batch2, cls_wt, cls_b2)


# speedup vs baseline: 1.0560x; 1.0560x over previous
import jax
import jax.numpy as jnp
from jax import lax
from jax.experimental import pallas as pl
from jax.experimental.pallas import tpu as pltpu
from jax.experimental.pallas import tpu_sc as plsc

_OUT = 200
_N = 10000
_E = 320000
_NG = 64
_DIN = 128
_LAYERS = 6
_H = 128
_AGGW = 2 * _H

_NC = 2
_NS = 16
_K = 64
_CS = 32
_NST = 10
_EPT = _K * _CS * _NST
_ACC = 10016
_TRASH = 10008
_STR = 624

_BR = 1000
_GB = _N // _BR


def _sc_body(m_hbm, src_hbm, dst_hbm, z_hbm, out_hbm,
             src_a, dst_a, src_b, dst_b, r0, r1, r2, r3, acc,
             si, s0, s1, s2, s3):
    cid = lax.axis_index("c")
    sid = lax.axis_index("s")

    @pl.when(sid < _NS - 1)
    def _():
        pltpu.sync_copy(z_hbm.at[pl.ds(sid * _STR, _STR)],
                        acc.at[pl.ds(sid * _STR, _STR)])

    @pl.when(sid == _NS - 1)
    def _():
        base = (_NS - 1) * _STR
        pltpu.sync_copy(z_hbm.at[pl.ds(base, _ACC - base)],
                        acc.at[pl.ds(base, _ACC - base)])

    pltpu.sync_copy(src_hbm.at[sid, 0], src_a)
    pltpu.sync_copy(dst_hbm.at[sid, 0], dst_a)
    plsc.subcore_barrier()

    mh = m_hbm.at[cid]
    idx_bufs = ((src_a, dst_a), (src_b, dst_b))
    rows = (r0, r1, r2, r3)
    sems = (s0, s1, s2, s3)

    for st in range(_NST):
        cur_s, cur_d = idx_bufs[st % 2]
        if st + 1 < _NST:
            nxt_s, nxt_d = idx_bufs[(st + 1) % 2]
            pltpu.async_copy(src_hbm.at[sid, st + 1], nxt_s, si)
            pltpu.async_copy(dst_hbm.at[sid, st + 1], nxt_d, si)

        for b in range(4):
            pltpu.async_copy(mh.at[cur_s.at[b]], rows[b], sems[b])

        @pl.loop(0, _CS - 4, step=4)
        def _(j, cur_s=cur_s, cur_d=cur_d):
            for b in range(4):
                pltpu.make_async_copy(mh.at[cur_s.at[j + b]],
                                      rows[b], sems[b]).wait()
                pltpu.sync_copy(rows[b], acc.at[cur_d.at[j + b]], add=True)
                pltpu.async_copy(mh.at[cur_s.at[j + b + 4]], rows[b], sems[b])

        for b in range(4):
            jj = _CS - 4 + b
            pltpu.make_async_copy(mh.at[cur_s.at[jj]],
                                  rows[b], sems[b]).wait()
            pltpu.sync_copy(rows[b], acc.at[cur_d.at[jj]], add=True)

        if st + 1 < _NST:
            pltpu.make_async_copy(src_hbm.at[sid, st + 1], nxt_s, si).wait()
            pltpu.make_async_copy(dst_hbm.at[sid, st + 1], nxt_d, si).wait()

    plsc.subcore_barrier()

    @pl.when(sid < _NS - 1)
    def _():
        pltpu.sync_copy(acc.at[pl.ds(sid * _STR, _STR)],
                        out_hbm.at[pl.ds(sid * _STR, _STR),
                                   pl.ds(cid * _H, _H)])

    @pl.when(sid == _NS - 1)
    def _():
        base = (_NS - 1) * _STR
        pltpu.sync_copy(acc.at[pl.ds(base, _N - base)],
                        out_hbm.at[pl.ds(base, _N - base),
                                   pl.ds(cid * _H, _H)])


def _sc_segment_sum(m3, src4, dst4, zeros):
    mesh = plsc.VectorSubcoreMesh(core_axis_name="c", subcore_axis_name="s")
    kern = pl.kernel(
        _sc_body,
        out_type=jax.ShapeDtypeStruct((_N, _AGGW), jnp.float32),
        mesh=mesh,
        scratch_types=[
            pltpu.VMEM((_CS, _K), jnp.int32),
            pltpu.VMEM((_CS, _K), jnp.int32),
            pltpu.VMEM((_CS, _K), jnp.int32),
            pltpu.VMEM((_CS, _K), jnp.int32),
            pltpu.VMEM((_K, _H), jnp.float32),
            pltpu.VMEM((_K, _H), jnp.float32),
            pltpu.VMEM((_K, _H), jnp.float32),
            pltpu.VMEM((_K, _H), jnp.float32),
            pltpu.VMEM_SHARED((_ACC, _H), jnp.float32),
            pltpu.SemaphoreType.DMA,
            pltpu.SemaphoreType.DMA,
            pltpu.SemaphoreType.DMA,
            pltpu.SemaphoreType.DMA,
            pltpu.SemaphoreType.DMA,
        ],
    )
    return kern(m3, src4, dst4, zeros)


def _dot(a, b):
    return jax.lax.dot_general(a, b, (((1,), (0,)), ((), ())),
                               preferred_element_type=jnp.float32)


def _full(shape):
    zeros = (0,) * len(shape)
    return pl.BlockSpec(shape, lambda i: zeros)


def _project0_body(x_ref, wa_ref, wb_ref, m_ref):
    x = x_ref[...]
    m_ref[0] = _dot(x, wa_ref[...])
    m_ref[1] = _dot(x, wb_ref[...])


def _project0(x, w0a, w0b):
    return pl.pallas_call(
        _project0_body,
        grid=(_GB,),
        in_specs=[pl.BlockSpec((_BR, _DIN), lambda i: (i, 0)),
                  _full((_DIN, _H)), _full((_DIN, _H))],
        out_specs=pl.BlockSpec((_NC, _BR, _H), lambda i: (0, i, 0)),
        out_shape=jax.ShapeDtypeStruct((_NC, _N, _H), jnp.float32),
        compiler_params=pltpu.CompilerParams(
            dimension_semantics=("arbitrary",)),
    )(x, w0a, w0b)


def _gru_math(p_ref, h_ref, wir, wiz, win, whr, whz, whn,
              bir, biz, bin_, bhr, bhz, bhn):
    agg = p_ref[...]
    h = h_ref[...]
    gir = _dot(agg, wir[...]) + bir[...]
    giz = _dot(agg, wiz[...]) + biz[...]
    gin = _dot(agg, win[...]) + bin_[...]
    ghr = _dot(h, whr[...]) + bhr[...]
    ghz = _dot(h, whz[...]) + bhz[...]
    ghn = _dot(h, whn[...]) + bhn[...]
    r = jax.nn.sigmoid(gir + ghr)
    z = jax.nn.sigmoid(giz + ghz)
    n = jnp.tanh(gin + r * ghn)
    return (1.0 - z) * n + z * h


def _gru_layer_body(p_ref, h_ref, wir, wiz, win, whr, whz, whn,
                    bir, biz, bin_, bhr, bhz, bhn, wna_ref, wnb_ref,
                    hout_ref, mout_ref):
    h_new = _gru_math(p_ref, h_ref, wir, wiz, win, whr, whz, whn,
                      bir, biz, bin_, bhr, bhz, bhn)
    hout_ref[...] = h_new
    mout_ref[0] = _dot(h_new, wna_ref[...])
    mout_ref[1] = _dot(h_new, wnb_ref[...])


def _gru_layer(p, h, mats, biases, w_next_a, w_next_b):
    in_specs = ([pl.BlockSpec((_BR, _AGGW), lambda i: (i, 0)),
                 pl.BlockSpec((_BR, _OUT), lambda i: (i, 0))]
                + [_full((_AGGW, _OUT))] * 3
                + [_full((_OUT, _OUT))] * 3
                + [_full((1, _OUT))] * 6
                + [_full((_OUT, _H))] * 2)
    return pl.pallas_call(
        _gru_layer_body,
        grid=(_GB,),
        in_specs=in_specs,
        out_specs=[pl.BlockSpec((_BR, _OUT), lambda i: (i, 0)),
                   pl.BlockSpec((_NC, _BR, _H), lambda i: (0, i, 0))],
        out_shape=[jax.ShapeDtypeStruct((_N, _OUT), jnp.float32),
                   jax.ShapeDtypeStruct((_NC, _N, _H), jnp.float32)],
        compiler_params=pltpu.CompilerParams(
            dimension_semantics=("arbitrary",)),
    )(p, h, *mats, *biases, w_next_a, w_next_b)


def _pool_body(h_ref, b_ref, cw_ref, cb_ref, o_ref, pooled_ref):
    i = pl.program_id(0)

    @pl.when(i == 0)
    def _():
        pooled_ref[...] = jnp.full((_NG, _OUT), -jnp.inf, jnp.float32)

    act = jnp.maximum(h_ref[...], 0.0)
    b = b_ref[...]
    g_lo = jnp.min(b)
    g_hi = jnp.max(b)

    def upd(g, _):
        mask = b == g
        mx = jnp.max(jnp.where(mask, act, -jnp.inf), axis=0, keepdims=True)
        cur = pooled_ref[pl.ds(g, 1), :]
        pooled_ref[pl.ds(g, 1), :] = jnp.maximum(cur, mx)
        return 0

    lax.fori_loop(g_lo, g_hi + 1, upd, 0)

    @pl.when(i == pl.num_programs(0) - 1)
    def _():
        logits = _dot(pooled_ref[...], cw_ref[...]) + cb_ref[...]
        o_ref[...] = jax.nn.sigmoid(logits)


def _pool(h, batch2, cls_wt, cls_b):
    in_specs = [pl.BlockSpec((_BR, _OUT), lambda i: (i, 0)),
                pl.BlockSpec((_BR, 1), lambda i: (i, 0)),
                _full((_OUT, 2)),
                _full((1, 2))]
    return pl.pallas_call(
        _pool_body,
        grid=(_GB,),
        in_specs=in_specs,
        out_specs=pl.BlockSpec((_NG, 2), lambda i: (0, 0)),
        out_shape=jax.ShapeDtypeStruct((_NG, 2), jnp.float32),
        scratch_shapes=[pltpu.VMEM((_NG, _OUT), jnp.float32)],
        compiler_params=pltpu.CompilerParams(
            dimension_semantics=("arbitrary",)),
    )(h, batch2, cls_wt, cls_b)


def _pad_cols(w, width):
    return jnp.pad(w, ((0, 0), (0, width - w.shape[1])))


def kernel(x, edge_index, batch, weight, w_ih, w_hh, b_ih, b_hh, cls_W, cls_b):
    pad = _EPT - _E // _NS
    src4 = jnp.pad(edge_index[0].reshape(_NS, _E // _NS),
                   ((0, 0), (0, pad))).reshape(_NS, _NST, _CS, _K)
    dst4 = jnp.pad(edge_index[1].reshape(_NS, _E // _NS),
                   ((0, 0), (0, pad)),
                   constant_values=_TRASH).reshape(_NS, _NST, _CS, _K)
    zeros = jnp.zeros((_ACC, _H), jnp.float32)
    h = jnp.pad(x, ((0, 0), (0, _OUT - _DIN)))

    wi = tuple(jnp.pad(w_ih[s * _OUT:(s + 1) * _OUT, :].T,
                       ((0, _AGGW - _OUT), (0, 0))) for s in range(3))
    wh = tuple(w_hh[s * _OUT:(s + 1) * _OUT, :].T for s in range(3))
    mats = wi + wh
    biases = tuple(bv[s * _OUT:(s + 1) * _OUT].reshape(1, _OUT)
                   for bv in (b_ih, b_hh) for s in range(3))
    batch2 = batch.reshape(_N, 1)
    cls_wt = cls_W.T
    cls_b2 = cls_b.reshape(1, 2)

    wa = [weight[i][:, 0:_H] for i in range(_LAYERS)]
    wb = [_pad_cols(weight[i][:, _H:_OUT], _H) for i in range(_LAYERS)]

    m = _project0(x, wa[0][0:_DIN], wb[0][0:_DIN])

    zw = jnp.zeros((_OUT, _H), jnp.float32)
    wa_next = jnp.stack(wa[1:] + [zw])
    wb_next = jnp.stack(wb[1:] + [zw])

    def step(carry, ws):
        h, m = carry
        w_next_a, w_next_b = ws
        p = _sc_segment_sum(m, src4, dst4, zeros)
        h, m = _gru_layer(p, h, mats, biases, w_next_a, w_next_b)
        return (h, m), None

    (h, m), _ = lax.scan(step, (h, m), (wa_next, wb_next))
    return _pool(h, batch2, cls_wt, cls_b2)
